# baseline (speedup 1.0000x reference)
import functools

import jax
import jax.numpy as jnp
import numpy as np
from jax import lax
from jax.experimental import pallas as pl
from jax.experimental.pallas import tpu as pltpu
from jax.experimental.pallas import tpu_sc as plsc

ROWS, COLS = 4096, 8192
LANES = 16
R = 2
WORDS = COLS // 4
BLOCKS = COLS // (4 * LANES)

_info = plsc.get_sparse_core_info()
NC, NS = _info.num_cores, _info.num_subcores
NW = NC * NS
ROWS_PER_W = ROWS // NW
GROUPS = ROWS_PER_W // R
HALF_GROUPS = GROUPS // 2

_BYTE_MASKS = tuple(np.uint32(0xFF) << np.uint32(8 * c) for c in range(4))


def _body(x_hbm, m_hbm, out_hbm, xb0, xb1, mb0, mb1, ob0, ob1,
          sin0, sin1, sout0, sout1):
    wid = lax.axis_index("s") * NC + lax.axis_index("c")
    base = wid * ROWS_PER_W
    xbs, mbs, obs = (xb0, xb1), (mb0, mb1), (sin0, sin1)
    del obs

    def start_load(g, slot):
        xb, mb, sin = (xb0, mb0, sin0) if slot == 0 else (xb1, mb1, sin1)
        row0 = base + g * R
        pltpu.async_copy(x_hbm.at[pl.ds(row0, R)], xb, sin)
        pltpu.async_copy(m_hbm.at[pl.ds(row0, R)], mb, sin)

    def wait_load(slot):
        xb, mb, sin = (xb0, mb0, sin0) if slot == 0 else (xb1, mb1, sin1)
        pltpu.make_async_copy(x_hbm.at[pl.ds(0, R)], xb, sin).wait()
        pltpu.make_async_copy(m_hbm.at[pl.ds(0, R)], mb, sin).wait()

    def start_store(g, slot):
        ob, sout = (ob0, sout0) if slot == 0 else (ob1, sout1)
        row0 = base + g * R
        pltpu.async_copy(ob, out_hbm.at[pl.ds(row0, R)], sout)

    def wait_store(slot):
        ob, sout = (ob0, sout0) if slot == 0 else (ob1, sout1)
        pltpu.make_async_copy(ob, out_hbm.at[pl.ds(0, R)], sout).wait()

    def compute(slot):
        xb, mb, ob = (xb0, mb0, ob0) if slot == 0 else (xb1, mb1, ob1)

        def block(j, carries):
            carries = list(carries)
            for r in range(R):
                w = mb[r, pl.ds(j * LANES, LANES)]
                for c in range(4):
                    off = j * 4 * LANES + c * LANES
                    xs = xb[r, pl.ds(off, LANES)]
                    bits = w & _BYTE_MASKS[c]
                    v = jnp.where(bits != jnp.uint32(0), xs, jnp.float32(0))
                    s = plsc.cumsum(v)
                    ob[r, pl.ds(off, LANES)] = s + carries[r]
                    carries[r] = carries[r] + s[LANES - 1]
            return tuple(carries)

        lax.fori_loop(0, BLOCKS, block, (jnp.float32(0),) * R, unroll=False)

    start_load(0, 0)
    start_load(1, 1)

    def ring(i, carry):
        for slot in range(2):
            g = i * 2 + slot
            wait_load(slot)

            @pl.when(i >= 1)
            def _():
                wait_store(slot)

            compute(slot)
            start_store(g, slot)

            @pl.when(i < HALF_GROUPS - 1)
            def _():
                start_load(g + 2, slot)

        return carry

    lax.fori_loop(0, HALF_GROUPS, ring, 0, unroll=False)
    wait_store(0)
    wait_store(1)


@jax.jit
def _masked_cumsum(x, mw):
    mesh = plsc.VectorSubcoreMesh(core_axis_name="c", subcore_axis_name="s")
    return pl.kernel(
        _body,
        out_type=jax.ShapeDtypeStruct((ROWS, COLS), jnp.float32),
        mesh=mesh,
        scratch_types=[
            pltpu.VMEM((R, COLS), jnp.float32),
            pltpu.VMEM((R, COLS), jnp.float32),
            pltpu.VMEM((R, WORDS), jnp.uint32),
            pltpu.VMEM((R, WORDS), jnp.uint32),
            pltpu.VMEM((R, COLS), jnp.float32),
            pltpu.VMEM((R, COLS), jnp.float32),
            pltpu.SemaphoreType.DMA,
            pltpu.SemaphoreType.DMA,
            pltpu.SemaphoreType.DMA,
            pltpu.SemaphoreType.DMA,
        ],
        compiler_params=pltpu.CompilerParams(needs_layout_passes=False),
    )(x, mw)


@jax.jit
def _pack_mask(mask):
    mu8 = mask.astype(jnp.uint8).reshape(ROWS, BLOCKS, 4, LANES)
    mu8 = mu8.swapaxes(-1, -2)
    mw = lax.bitcast_convert_type(mu8, jnp.uint32)
    return mw.reshape(ROWS, WORDS)


def kernel(x, mask):
    return _masked_cumsum(x, _pack_mask(mask))


# speedup vs baseline: 1.0751x; 1.0751x over previous
import functools

import jax
import jax.numpy as jnp
import numpy as np
from jax import lax
from jax.experimental import pallas as pl
from jax.experimental.pallas import tpu as pltpu
from jax.experimental.pallas import tpu_sc as plsc

ROWS, COLS = 4096, 8192
LANES = 16
R = 4
WORDS = COLS // 4
BLOCKS = COLS // (4 * LANES)

_info = plsc.get_sparse_core_info()
NC, NS = _info.num_cores, _info.num_subcores
NW = NC * NS
ROWS_PER_W = ROWS // NW
GROUPS = ROWS_PER_W // R
NSLOT = 3

_BYTE_MASKS = tuple(np.uint32(0xFF) << np.uint32(8 * c) for c in range(4))


def _body(x_hbm, m_hbm, out_hbm,
          xb0, xb1, xb2, mb0, mb1, mb2,
          sin0, sin1, sin2, sout0, sout1, sout2):
    wid = lax.axis_index("s") * NC + lax.axis_index("c")
    base = wid * ROWS_PER_W
    xbs = (xb0, xb1, xb2)
    mbs = (mb0, mb1, mb2)
    sins = (sin0, sin1, sin2)
    souts = (sout0, sout1, sout2)
    idx15 = jnp.full((LANES, 1), LANES - 1, jnp.int32)
    gd = lax.GatherDimensionNumbers(
        offset_dims=(), collapsed_slice_dims=(0,), start_index_map=(0,))

    def bcast_last(s):
        return lax.gather(s, idx15, gd, (1,),
                          mode=lax.GatherScatterMode.PROMISE_IN_BOUNDS)

    def start_load(g, slot):
        row0 = base + g * R
        pltpu.async_copy(x_hbm.at[pl.ds(row0, R)], xbs[slot], sins[slot])
        pltpu.async_copy(m_hbm.at[pl.ds(row0, R)], mbs[slot], sins[slot])

    def wait_load(slot):
        pltpu.make_async_copy(x_hbm.at[pl.ds(0, R)], xbs[slot], sins[slot]).wait()
        pltpu.make_async_copy(m_hbm.at[pl.ds(0, R)], mbs[slot], sins[slot]).wait()

    def start_store(g, slot):
        row0 = base + g * R
        pltpu.async_copy(xbs[slot], out_hbm.at[pl.ds(row0, R)], souts[slot])

    def wait_store(slot):
        pltpu.make_async_copy(xbs[slot], out_hbm.at[pl.ds(0, R)], souts[slot]).wait()

    def compute(slot):
        xb, mb = xbs[slot], mbs[slot]

        def block(j, carries):
            carries = list(carries)
            for r in range(R):
                w = mb[r, pl.ds(j * LANES, LANES)]
                for c in range(4):
                    off = j * 4 * LANES + c * LANES
                    xs = xb[r, pl.ds(off, LANES)]
                    bits = w & _BYTE_MASKS[c]
                    v = jnp.where(bits != jnp.uint32(0), xs, jnp.float32(0))
                    s = plsc.cumsum(v) + carries[r]
                    xb[r, pl.ds(off, LANES)] = s
                    carries[r] = bcast_last(s)
            return tuple(carries)

        zero = jnp.zeros((LANES,), jnp.float32)
        lax.fori_loop(0, BLOCKS, block, (zero,) * R, unroll=False)

    def step(g, slot, *, traced):
        when = pl.when if traced else (lambda p: (lambda f: f() if p else None))
        nxt = (slot + 1) % NSLOT

        @when(g >= 2 if not traced else g >= 2)
        def _w():
            wait_store(nxt)

        @when(g < GROUPS - 1 if not traced else g < GROUPS - 1)
        def _l():
            start_load(g + 1, nxt)

        wait_load(slot)
        compute(slot)
        start_store(g, slot)

    start_load(0, 0)

    def ring(i, carry):
        for k in range(NSLOT):
            step(i * NSLOT + k, k, traced=True)
        return carry

    main_iters = GROUPS // NSLOT
    lax.fori_loop(0, main_iters, ring, 0, unroll=False)
    for g in range(main_iters * NSLOT, GROUPS):
        step(g, g % NSLOT, traced=False)
    wait_store((GROUPS - 2) % NSLOT)
    wait_store((GROUPS - 1) % NSLOT)


@jax.jit
def _masked_cumsum(x, mw):
    mesh = plsc.VectorSubcoreMesh(core_axis_name="c", subcore_axis_name="s")
    return pl.kernel(
        _body,
        out_type=jax.ShapeDtypeStruct((ROWS, COLS), jnp.float32),
        mesh=mesh,
        scratch_types=[
            pltpu.VMEM((R, COLS), jnp.float32),
            pltpu.VMEM((R, COLS), jnp.float32),
            pltpu.VMEM((R, COLS), jnp.float32),
            pltpu.VMEM((R, WORDS), jnp.uint32),
            pltpu.VMEM((R, WORDS), jnp.uint32),
            pltpu.VMEM((R, WORDS), jnp.uint32),
            pltpu.SemaphoreType.DMA,
            pltpu.SemaphoreType.DMA,
            pltpu.SemaphoreType.DMA,
            pltpu.SemaphoreType.DMA,
            pltpu.SemaphoreType.DMA,
            pltpu.SemaphoreType.DMA,
        ],
        compiler_params=pltpu.CompilerParams(needs_layout_passes=False),
    )(x, mw)


@jax.jit
def _pack_mask(mask):
    mu8 = mask.astype(jnp.uint8).reshape(ROWS, BLOCKS, 4, LANES)
    mu8 = mu8.swapaxes(-1, -2)
    mw = lax.bitcast_convert_type(mu8, jnp.uint32)
    return mw.reshape(ROWS, WORDS)


def kernel(x, mask):
    return _masked_cumsum(x, _pack_mask(mask))


# speedup vs baseline: 1.0778x; 1.0025x over previous
import functools

import jax
import jax.numpy as jnp
import numpy as np
from jax import lax
from jax.experimental import pallas as pl
from jax.experimental.pallas import tpu as pltpu
from jax.experimental.pallas import tpu_sc as plsc

ROWS, COLS = 4096, 8192
LANES = 16
R = 4
WORDS = COLS // 4
BLOCKS = COLS // (4 * LANES)

_info = plsc.get_sparse_core_info()
NC, NS = _info.num_cores, _info.num_subcores
NW = NC * NS
ROWS_PER_W = ROWS // NW
GROUPS = ROWS_PER_W // R
NSLOT = 3

_BYTE_MASKS = tuple(np.uint32(0xFF) << np.uint32(8 * c) for c in range(4))


def _body(x_hbm, m_hbm, out_hbm,
          xb0, xb1, xb2, mb0, mb1, mb2,
          sin0, sin1, sin2, sout0, sout1, sout2):
    wid = lax.axis_index("s") * NC + lax.axis_index("c")
    base = wid * ROWS_PER_W
    xbs = (xb0, xb1, xb2)
    mbs = (mb0, mb1, mb2)
    sins = (sin0, sin1, sin2)
    souts = (sout0, sout1, sout2)

    def start_load(g, slot):
        row0 = base + g * R
        pltpu.async_copy(x_hbm.at[pl.ds(row0, R)], xbs[slot], sins[slot])
        pltpu.async_copy(m_hbm.at[pl.ds(row0, R)], mbs[slot], sins[slot])

    def wait_load(slot):
        pltpu.make_async_copy(x_hbm.at[pl.ds(0, R)], xbs[slot], sins[slot]).wait()
        pltpu.make_async_copy(m_hbm.at[pl.ds(0, R)], mbs[slot], sins[slot]).wait()

    def start_store(g, slot):
        row0 = base + g * R
        pltpu.async_copy(xbs[slot], out_hbm.at[pl.ds(row0, R)], souts[slot])

    def wait_store(slot):
        pltpu.make_async_copy(xbs[slot], out_hbm.at[pl.ds(0, R)], souts[slot]).wait()

    def compute(slot):
        xb, mb = xbs[slot], mbs[slot]

        def block(j, carries):
            carries = list(carries)
            for r in range(R):
                w = mb[r, pl.ds(j * LANES, LANES)]
                for c in range(4):
                    off = j * 4 * LANES + c * LANES
                    xs = xb[r, pl.ds(off, LANES)]
                    bits = w & _BYTE_MASKS[c]
                    v = jnp.where(bits != jnp.uint32(0), xs, jnp.float32(0))
                    s = plsc.cumsum(v) + carries[r]
                    xb[r, pl.ds(off, LANES)] = s
                    carries[r] = s[LANES - 1]
            return tuple(carries)

        lax.fori_loop(0, BLOCKS, block, (jnp.float32(0),) * R, unroll=False)

    def step(g, slot, *, traced):
        when = pl.when if traced else (lambda p: (lambda f: f() if p else None))
        nxt = (slot + 1) % NSLOT

        @when(g >= 2 if not traced else g >= 2)
        def _w():
            wait_store(nxt)

        @when(g < GROUPS - 1 if not traced else g < GROUPS - 1)
        def _l():
            start_load(g + 1, nxt)

        wait_load(slot)
        compute(slot)
        start_store(g, slot)

    start_load(0, 0)

    def ring(i, carry):
        for k in range(NSLOT):
            step(i * NSLOT + k, k, traced=True)
        return carry

    main_iters = GROUPS // NSLOT
    lax.fori_loop(0, main_iters, ring, 0, unroll=False)
    for g in range(main_iters * NSLOT, GROUPS):
        step(g, g % NSLOT, traced=False)
    wait_store((GROUPS - 2) % NSLOT)
    wait_store((GROUPS - 1) % NSLOT)


@jax.jit
def _masked_cumsum(x, mw):
    mesh = plsc.VectorSubcoreMesh(core_axis_name="c", subcore_axis_name="s")
    return pl.kernel(
        _body,
        out_type=jax.ShapeDtypeStruct((ROWS, COLS), jnp.float32),
        mesh=mesh,
        scratch_types=[
            pltpu.VMEM((R, COLS), jnp.float32),
            pltpu.VMEM((R, COLS), jnp.float32),
            pltpu.VMEM((R, COLS), jnp.float32),
            pltpu.VMEM((R, WORDS), jnp.uint32),
            pltpu.VMEM((R, WORDS), jnp.uint32),
            pltpu.VMEM((R, WORDS), jnp.uint32),
            pltpu.SemaphoreType.DMA,
            pltpu.SemaphoreType.DMA,
            pltpu.SemaphoreType.DMA,
            pltpu.SemaphoreType.DMA,
            pltpu.SemaphoreType.DMA,
            pltpu.SemaphoreType.DMA,
        ],
        compiler_params=pltpu.CompilerParams(needs_layout_passes=False),
    )(x, mw)


@jax.jit
def _pack_mask(mask):
    mu8 = mask.astype(jnp.uint8).reshape(ROWS, BLOCKS, 4, LANES)
    mu8 = mu8.swapaxes(-1, -2)
    mw = lax.bitcast_convert_type(mu8, jnp.uint32)
    return mw.reshape(ROWS, WORDS)


def kernel(x, mask):
    return _masked_cumsum(x, _pack_mask(mask))
